# baseline (speedup 1.0000x reference)
import functools

import jax
import jax.numpy as jnp
from jax import lax
from jax.experimental import pallas as pl
from jax.experimental.pallas import tpu as pltpu
from jax.experimental.pallas import tpu_sc as plsc

_B = 16384
_D = 64
_NC = 2
_NS = 16
_NW = _NC * _NS
_BPW = _B // _NW
_L = 16


def _mf_body(uid_hbm, mid_hbm, uemb_hbm, memb_hbm, ubias_hbm, mbias_hbm,
             out_hbm,
             uid_v, mid_v, urows_v, mrows_v, ubias_v, mbias_v, out_v,
             sem_u, sem_m, sem_bu, sem_bm):
    wid = lax.axis_index("s") * _NC + lax.axis_index("c")
    base = wid * _BPW

    pltpu.sync_copy(uid_hbm.at[pl.ds(base, _BPW)], uid_v)
    pltpu.sync_copy(mid_hbm.at[pl.ds(base, _BPW)], mid_v)

    cu = pltpu.async_copy(uemb_hbm.at[uid_v], urows_v, sem_u)
    cm = pltpu.async_copy(memb_hbm.at[mid_v], mrows_v, sem_m)
    cbu = pltpu.async_copy(ubias_hbm.at[uid_v], ubias_v, sem_bu)
    cbm = pltpu.async_copy(mbias_hbm.at[mid_v], mbias_v, sem_bm)
    cbu.wait()
    cbm.wait()
    cu.wait()
    cm.wait()

    def group(g, carry):
        row0 = g * _L
        rows = lax.iota(jnp.int32, _L) + row0
        acc = ubias_v[pl.ds(row0, _L)] + mbias_v[pl.ds(row0, _L)]
        for d in range(_D):
            dd = jnp.full((_L,), d, jnp.int32)
            u = plsc.load_gather(urows_v, [rows, dd])
            m = plsc.load_gather(mrows_v, [rows, dd])
            acc = acc + u * m
        out_v[pl.ds(row0, _L)] = acc
        return carry

    lax.fori_loop(0, _BPW // _L, group, 0)
    pltpu.sync_copy(out_v, out_hbm.at[pl.ds(base, _BPW)])


@functools.partial(jax.jit, donate_argnums=())
def kernel(user_ids, movie_ids, user_emb, movie_emb, user_bias, movie_bias):
    run = pl.kernel(
        _mf_body,
        out_type=jax.ShapeDtypeStruct((_B,), jnp.float32),
        mesh=plsc.VectorSubcoreMesh(core_axis_name="c", subcore_axis_name="s"),
        compiler_params=pltpu.CompilerParams(
            needs_layout_passes=False, use_tc_tiling_on_sc=False),
        scratch_types=[
            pltpu.VMEM((_BPW,), jnp.int32),
            pltpu.VMEM((_BPW,), jnp.int32),
            pltpu.VMEM((_BPW, _D), jnp.float32),
            pltpu.VMEM((_BPW, _D), jnp.float32),
            pltpu.VMEM((_BPW,), jnp.float32),
            pltpu.VMEM((_BPW,), jnp.float32),
            pltpu.VMEM((_BPW,), jnp.float32),
            pltpu.SemaphoreType.DMA,
            pltpu.SemaphoreType.DMA,
            pltpu.SemaphoreType.DMA,
            pltpu.SemaphoreType.DMA,
        ],
    )
    return run(user_ids.astype(jnp.int32), movie_ids.astype(jnp.int32),
               user_emb, movie_emb,
               user_bias.reshape(-1), movie_bias.reshape(-1))
